# baseline (speedup 1.0000x reference)
import functools

import jax
import jax.numpy as jnp
import numpy as np
from jax.experimental import pallas as pl
from jax.experimental.pallas import tpu as pltpu

N0, N1, N2 = 64 * 64, 32 * 32, 16 * 16
N = N0 + N1 + N2
G = 20
C = 80
BINS = 16
TOPK = 10
EPS = 1e-9
PB = 2
W_CLS, W_IOU, W_DFL = 0.5, 7.5, 1.5


def _make_geo():
    rows = []
    for (h, w, s) in ((64, 64, 8.0), (32, 32, 16.0), (16, 16, 32.0)):
        yy, xx = np.meshgrid(np.arange(h, dtype=np.float32),
                             np.arange(w, dtype=np.float32), indexing="ij")
        cx = ((xx + 0.5) * s).reshape(-1)
        cy = ((yy + 0.5) * s).reshape(-1)
        st = np.full(h * w, s, np.float32)
        rows.append(np.stack([cx, cy, st], 0))
    geo = np.concatenate(rows, axis=1)
    return np.concatenate([geo, np.zeros((5, N), np.float32)], axis=0)


_GEO = _make_geo()

_ATAN_C = (0.9999999937538802, -0.33333137974716015, 0.19993694319366187,
           -0.14211106054436182, 0.10667486902233639, -0.07556900202632058,
           0.043278241746605356, -0.01641319040050145, 0.0029327619377836774)


def _atan(x):
    ax = jnp.abs(x)
    inv = ax > 1.0
    t = jnp.where(inv, 1.0 / jnp.maximum(ax, 1e-30), ax)
    u = t * t
    p = jnp.float32(_ATAN_C[-1])
    for c in _ATAN_C[-2::-1]:
        p = p * u + c
    r = t * p
    r = jnp.where(inv, jnp.float32(np.pi / 2) - r, r)
    return jnp.where(x < 0, -r, r)


def _mm(a, b):
    return jnp.dot(a, b, preferred_element_type=jnp.float32)


def _one_image(cx, cy, stride, x, d, gtb, gtbt, lblraw):
    bsel = jax.lax.broadcasted_iota(jnp.int32, (8, 64), 0)
    bcol = jax.lax.broadcasted_iota(jnp.int32, (8, 64), 1)
    side_of = bcol // BINS
    binv = (bcol % BINS).astype(jnp.float32)
    smat = jnp.where((bsel % 4) == side_of,
                     jnp.where(bsel < 4, 1.0, binv), 0.0)
    e = jnp.exp(d)
    red = _mm(smat, e)
    se4 = red[0:4, :]
    dist4 = red[4:8, :] / se4 * stride
    logz4 = jnp.log(se4)
    bx1 = cx - dist4[0:1, :]
    by1 = cy - dist4[1:2, :]
    bx2 = cx + dist4[2:3, :]
    by2 = cy + dist4[3:4, :]

    gx1 = gtb[:, 0:1]
    gy1 = gtb[:, 1:2]
    gx2 = gtb[:, 2:3]
    gy2 = gtb[:, 3:4]
    lbl = jnp.clip(lblraw, 0, C - 1)

    lbl_oh = (lbl == jax.lax.broadcasted_iota(jnp.int32, (G, C), 1)
              ).astype(jnp.float32)
    cls_g = _mm(lbl_oh, x)
    cls_s = jax.nn.sigmoid(cls_g)

    ix1 = jnp.maximum(gx1, bx1)
    iy1 = jnp.maximum(gy1, by1)
    ix2 = jnp.minimum(gx2, bx2)
    iy2 = jnp.minimum(gy2, by2)
    inter = jnp.clip(ix2 - ix1, 0) * jnp.clip(iy2 - iy1, 0)
    ag = (gx2 - gx1) * (gy2 - gy1)
    ap = (bx2 - bx1) * (by2 - by1)
    union = ag + ap - inter
    iou = inter / (union + EPS)

    iou2 = iou * iou
    iou6 = iou2 * iou2 * iou2
    align = jnp.sqrt(cls_s) * iou6

    pcx = (bx1 + bx2) * 0.5
    pcy = (by1 + by2) * 0.5
    in_gt = (pcx >= gx1) & (pcx < gx2) & (pcy >= gy1) & (pcy < gy2)
    valid = ((gx2 - gx1) > 0) & ((gy2 - gy1) > 0)
    mask = in_gt & valid
    metric = jnp.where(mask, align, 0.0)

    mwork = metric
    mv = None
    for _ in range(TOPK):
        mv = jnp.max(mwork, axis=1, keepdims=True)
        mwork = jnp.where(mwork == mv, -1.0, mwork)
    mp = ((metric >= mv) & (metric > EPS)).astype(jnp.float32)

    ones_g = jnp.ones((1, G), jnp.float32)
    g_iota = jax.lax.broadcasted_iota(jnp.int32, (G, N), 0)
    g_col = jax.lax.broadcasted_iota(jnp.int32, (G, G), 1)
    g_row = jax.lax.broadcasted_iota(jnp.int32, (G, G), 0)
    tril = (g_col <= g_row).astype(jnp.float32)

    fg_count = _mm(ones_g, mp)

    x_iou = jnp.where(mp > 0, iou, -1.0)
    mxv = jnp.max(x_iou, axis=0, keepdims=True)
    eqm = (x_iou == mxv).astype(jnp.float32)
    is_max = jnp.where(_mm(tril, eqm) == 1.0, eqm, 0.0)
    mp = jnp.where(fg_count > 1, is_max * mp, mp)

    fg_f = jnp.minimum(_mm(ones_g, mp), 1.0)
    cum = _mm(tril, mp)
    moh = jnp.where((mp > 0) & (cum == 1.0), 1.0, 0.0)
    moh = moh + (1.0 - fg_f) * (g_iota == 0).astype(jnp.float32)

    tboxes = _mm(gtbt, moh)
    tx1 = tboxes[0:1, :]
    ty1 = tboxes[1:2, :]
    tx2 = tboxes[2:3, :]
    ty2 = tboxes[3:4, :]
    xsel = _mm(ones_g, moh * cls_g)

    align_m = align * mp
    pos_align = jnp.max(align_m, axis=1, keepdims=True)
    pos_iou = jnp.max(iou * mp, axis=1, keepdims=True)
    norm = jnp.max(align_m * pos_iou / (pos_align + EPS), axis=0,
                   keepdims=True)
    w = norm * fg_f

    ones_c = jnp.ones((1, C), jnp.float32)
    t = jnp.abs(x)
    l1p = jnp.log1p(jnp.exp(-t))
    base = (0.5 * jnp.sum(_mm(ones_c, x + t))
            + jnp.sum(_mm(ones_c, l1p)))
    cls_sum = base - jnp.sum(w * xsel)

    cix1 = jnp.maximum(bx1, tx1)
    ciy1 = jnp.maximum(by1, ty1)
    cix2 = jnp.minimum(bx2, tx2)
    ciy2 = jnp.minimum(by2, ty2)
    cinter = jnp.clip(cix2 - cix1, 0) * jnp.clip(ciy2 - ciy1, 0)
    at = (tx2 - tx1) * (ty2 - ty1)
    cunion = ap + at - cinter
    ciou = cinter / (cunion + EPS)
    ex1 = jnp.minimum(bx1, tx1)
    ey1 = jnp.minimum(by1, ty1)
    ex2 = jnp.maximum(bx2, tx2)
    ey2 = jnp.maximum(by2, ty2)
    c2 = (ex2 - ex1) ** 2 + (ey2 - ey1) ** 2 + EPS
    rho2 = ((bx1 + bx2 - tx1 - tx2) ** 2 + (by1 + by2 - ty1 - ty2) ** 2) / 4.0
    wp = bx2 - bx1
    hp = by2 - by1 + EPS
    wt = tx2 - tx1
    ht = ty2 - ty1 + EPS
    v = (4.0 / (np.pi ** 2)) * (_atan(wt / ht) - _atan(wp / hp)) ** 2
    a = v / (v - ciou + 1.0 + EPS)
    lci = 1.0 - (ciou - rho2 / c2 - a * v)
    iou_sum = jnp.sum(lci * fg_f)

    tdists = (jnp.clip(cx - tx1, 0), jnp.clip(cy - ty1, 0),
              jnp.clip(tx2 - cx, 0), jnp.clip(ty2 - cy, 0))
    b_iota = jax.lax.broadcasted_iota(
        jnp.int32, (BINS, N), 0).astype(jnp.float32)
    ones_b = jnp.ones((1, BINS), jnp.float32)
    val = jnp.zeros((1, N), jnp.float32)
    for s in range(4):
        tb = jnp.clip(tdists[s], 0.0, BINS - 1 - 1e-6)
        blk = d[BINS * s:BINS * (s + 1), :]
        tw = jnp.maximum(1.0 - jnp.abs(tb - b_iota), 0.0)
        val = val + _mm(ones_b, tw * blk)
    dfl_nll = (logz4[0:1, :] + logz4[1:2, :]
               + logz4[2:3, :] + logz4[3:4, :]) - val
    dfl_sum = jnp.sum(fg_f * dfl_nll)

    np_sum = jnp.sum(fg_f)

    return iou_sum, cls_sum, dfl_sum, np_sum


def _loss_kernel(geo_ref, p0_ref, p1_ref, p2_ref, gtb_ref, gtbt_ref,
                 gtl_ref, out_ref):
    cx = geo_ref[0:1, :]
    cy = geo_ref[1:2, :]
    stride = geo_ref[2:3, :]
    for i in range(PB):
        x = jnp.concatenate(
            [p0_ref[i, 64:64 + C, :], p1_ref[i, 64:64 + C, :],
             p2_ref[i, 64:64 + C, :]], axis=1)
        d = jnp.concatenate(
            [p0_ref[i, 0:64, :], p1_ref[i, 0:64, :],
             p2_ref[i, 0:64, :]], axis=1)
        sums = _one_image(cx, cy, stride, x, d, gtb_ref[i], gtbt_ref[i],
                          gtl_ref[i])
        out_ref[i, 0, 0] = sums[0]
        out_ref[i, 0, 1] = sums[1]
        out_ref[i, 0, 2] = sums[2]
        out_ref[i, 0, 3] = sums[3]


@functools.partial(jax.jit, static_argnames=())
def kernel(p0, p1, p2, gt_bboxes, gt_labels):
    B = p0.shape[0]
    p0r = p0.reshape(B, 144, N0)
    p1r = p1.reshape(B, 144, N1)
    p2r = p2.reshape(B, 144, N2)
    gtbt = gt_bboxes.transpose(0, 2, 1)
    gtl = gt_labels.astype(jnp.int32).reshape(B, G, 1)
    geo = jnp.asarray(_GEO)

    parts = pl.pallas_call(
        _loss_kernel,
        grid=(B // PB,),
        in_specs=[
            pl.BlockSpec((8, N), lambda b: (0, 0)),
            pl.BlockSpec((PB, 144, N0), lambda b: (b, 0, 0)),
            pl.BlockSpec((PB, 144, N1), lambda b: (b, 0, 0)),
            pl.BlockSpec((PB, 144, N2), lambda b: (b, 0, 0)),
            pl.BlockSpec((PB, G, 4), lambda b: (b, 0, 0)),
            pl.BlockSpec((PB, 4, G), lambda b: (b, 0, 0)),
            pl.BlockSpec((PB, G, 1), lambda b: (b, 0, 0)),
        ],
        out_specs=pl.BlockSpec((PB, 1, 4), lambda b: (b, 0, 0),
                               memory_space=pltpu.SMEM),
        out_shape=jax.ShapeDtypeStruct((B, 1, 4), jnp.float32),
        compiler_params=pltpu.CompilerParams(
            dimension_semantics=("parallel",)),
    )(geo, p0r, p1r, p2r, gt_bboxes, gtbt, gtl)

    sums = parts.sum(axis=(0, 1))
    denom = jnp.maximum(1.0, sums[3])
    return jnp.stack([W_IOU * sums[0], W_CLS * sums[1],
                      W_DFL * sums[2]]) / denom


# speedup vs baseline: 1.0087x; 1.0087x over previous
import functools

import jax
import jax.numpy as jnp
import numpy as np
from jax.experimental import pallas as pl
from jax.experimental.pallas import tpu as pltpu

N0, N1, N2 = 64 * 64, 32 * 32, 16 * 16
N = N0 + N1 + N2
G = 20
C = 80
BINS = 16
TOPK = 10
EPS = 1e-9
PB = 1
W_CLS, W_IOU, W_DFL = 0.5, 7.5, 1.5


def _make_geo():
    rows = []
    for (h, w, s) in ((64, 64, 8.0), (32, 32, 16.0), (16, 16, 32.0)):
        yy, xx = np.meshgrid(np.arange(h, dtype=np.float32),
                             np.arange(w, dtype=np.float32), indexing="ij")
        cx = ((xx + 0.5) * s).reshape(-1)
        cy = ((yy + 0.5) * s).reshape(-1)
        st = np.full(h * w, s, np.float32)
        rows.append(np.stack([cx, cy, st], 0))
    geo = np.concatenate(rows, axis=1)
    return np.concatenate([geo, np.zeros((5, N), np.float32)], axis=0)


_GEO = _make_geo()

_ATAN_C = (0.9999999937538802, -0.33333137974716015, 0.19993694319366187,
           -0.14211106054436182, 0.10667486902233639, -0.07556900202632058,
           0.043278241746605356, -0.01641319040050145, 0.0029327619377836774)


def _atan(x):
    ax = jnp.abs(x)
    inv = ax > 1.0
    t = jnp.where(inv, 1.0 / jnp.maximum(ax, 1e-30), ax)
    u = t * t
    p = jnp.float32(_ATAN_C[-1])
    for c in _ATAN_C[-2::-1]:
        p = p * u + c
    r = t * p
    r = jnp.where(inv, jnp.float32(np.pi / 2) - r, r)
    return jnp.where(x < 0, -r, r)


def _mm(a, b):
    return jnp.dot(a, b, preferred_element_type=jnp.float32)


def _one_image(cx, cy, stride, x, d, gtb, gtbt, lblraw):
    bsel = jax.lax.broadcasted_iota(jnp.int32, (8, 64), 0)
    bcol = jax.lax.broadcasted_iota(jnp.int32, (8, 64), 1)
    side_of = bcol // BINS
    binv = (bcol % BINS).astype(jnp.float32)
    smat = jnp.where((bsel % 4) == side_of,
                     jnp.where(bsel < 4, 1.0, binv), 0.0)
    e = jnp.exp(d)
    red = _mm(smat, e)
    se4 = red[0:4, :]
    dist4 = red[4:8, :] / se4 * stride
    logz4 = jnp.log(se4)
    bx1 = cx - dist4[0:1, :]
    by1 = cy - dist4[1:2, :]
    bx2 = cx + dist4[2:3, :]
    by2 = cy + dist4[3:4, :]

    gx1 = gtb[:, 0:1]
    gy1 = gtb[:, 1:2]
    gx2 = gtb[:, 2:3]
    gy2 = gtb[:, 3:4]
    lbl = jnp.clip(lblraw, 0, C - 1)

    lbl_oh = (lbl == jax.lax.broadcasted_iota(jnp.int32, (G, C), 1)
              ).astype(jnp.float32)
    cls_g = _mm(lbl_oh, x)
    cls_s = jax.nn.sigmoid(cls_g)

    ix1 = jnp.maximum(gx1, bx1)
    iy1 = jnp.maximum(gy1, by1)
    ix2 = jnp.minimum(gx2, bx2)
    iy2 = jnp.minimum(gy2, by2)
    inter = jnp.clip(ix2 - ix1, 0) * jnp.clip(iy2 - iy1, 0)
    ag = (gx2 - gx1) * (gy2 - gy1)
    ap = (bx2 - bx1) * (by2 - by1)
    union = ag + ap - inter
    iou = inter / (union + EPS)

    iou2 = iou * iou
    iou6 = iou2 * iou2 * iou2
    align = jnp.sqrt(cls_s) * iou6

    pcx = (bx1 + bx2) * 0.5
    pcy = (by1 + by2) * 0.5
    in_gt = (pcx >= gx1) & (pcx < gx2) & (pcy >= gy1) & (pcy < gy2)
    valid = ((gx2 - gx1) > 0) & ((gy2 - gy1) > 0)
    mask = in_gt & valid
    metric = jnp.where(mask, align, 0.0)

    mwork = metric
    mv = None
    for _ in range(TOPK):
        mv = jnp.max(mwork, axis=1, keepdims=True)
        mwork = jnp.where(mwork == mv, -1.0, mwork)
    mp = ((metric >= mv) & (metric > EPS)).astype(jnp.float32)

    ones_g = jnp.ones((1, G), jnp.float32)
    g_iota = jax.lax.broadcasted_iota(jnp.int32, (G, N), 0)
    g_col = jax.lax.broadcasted_iota(jnp.int32, (G, G), 1)
    g_row = jax.lax.broadcasted_iota(jnp.int32, (G, G), 0)
    tril = (g_col <= g_row).astype(jnp.float32)

    fg_count = _mm(ones_g, mp)

    x_iou = jnp.where(mp > 0, iou, -1.0)
    mxv = jnp.max(x_iou, axis=0, keepdims=True)
    eqm = (x_iou == mxv).astype(jnp.float32)
    is_max = jnp.where(_mm(tril, eqm) == 1.0, eqm, 0.0)
    mp = jnp.where(fg_count > 1, is_max * mp, mp)

    fg_f = jnp.minimum(_mm(ones_g, mp), 1.0)
    cum = _mm(tril, mp)
    moh = jnp.where((mp > 0) & (cum == 1.0), 1.0, 0.0)
    moh = moh + (1.0 - fg_f) * (g_iota == 0).astype(jnp.float32)

    tboxes = _mm(gtbt, moh)
    tx1 = tboxes[0:1, :]
    ty1 = tboxes[1:2, :]
    tx2 = tboxes[2:3, :]
    ty2 = tboxes[3:4, :]
    xsel = _mm(ones_g, moh * cls_g)

    align_m = align * mp
    pos_align = jnp.max(align_m, axis=1, keepdims=True)
    pos_iou = jnp.max(iou * mp, axis=1, keepdims=True)
    norm = jnp.max(align_m * pos_iou / (pos_align + EPS), axis=0,
                   keepdims=True)
    w = norm * fg_f

    ones_c = jnp.ones((1, C), jnp.float32)
    t = jnp.abs(x)
    l1p = jnp.log1p(jnp.exp(-t))
    base = (0.5 * jnp.sum(_mm(ones_c, x + t))
            + jnp.sum(_mm(ones_c, l1p)))
    cls_sum = base - jnp.sum(w * xsel)

    cix1 = jnp.maximum(bx1, tx1)
    ciy1 = jnp.maximum(by1, ty1)
    cix2 = jnp.minimum(bx2, tx2)
    ciy2 = jnp.minimum(by2, ty2)
    cinter = jnp.clip(cix2 - cix1, 0) * jnp.clip(ciy2 - ciy1, 0)
    at = (tx2 - tx1) * (ty2 - ty1)
    cunion = ap + at - cinter
    ciou = cinter / (cunion + EPS)
    ex1 = jnp.minimum(bx1, tx1)
    ey1 = jnp.minimum(by1, ty1)
    ex2 = jnp.maximum(bx2, tx2)
    ey2 = jnp.maximum(by2, ty2)
    c2 = (ex2 - ex1) ** 2 + (ey2 - ey1) ** 2 + EPS
    rho2 = ((bx1 + bx2 - tx1 - tx2) ** 2 + (by1 + by2 - ty1 - ty2) ** 2) / 4.0
    wp = bx2 - bx1
    hp = by2 - by1 + EPS
    wt = tx2 - tx1
    ht = ty2 - ty1 + EPS
    v = (4.0 / (np.pi ** 2)) * (_atan(wt / ht) - _atan(wp / hp)) ** 2
    a = v / (v - ciou + 1.0 + EPS)
    lci = 1.0 - (ciou - rho2 / c2 - a * v)
    iou_sum = jnp.sum(lci * fg_f)

    tdists = (jnp.clip(cx - tx1, 0), jnp.clip(cy - ty1, 0),
              jnp.clip(tx2 - cx, 0), jnp.clip(ty2 - cy, 0))
    b_iota = jax.lax.broadcasted_iota(
        jnp.int32, (BINS, N), 0).astype(jnp.float32)
    ones_b = jnp.ones((1, BINS), jnp.float32)
    val = jnp.zeros((1, N), jnp.float32)
    for s in range(4):
        tb = jnp.clip(tdists[s], 0.0, BINS - 1 - 1e-6)
        blk = d[BINS * s:BINS * (s + 1), :]
        tw = jnp.maximum(1.0 - jnp.abs(tb - b_iota), 0.0)
        val = val + _mm(ones_b, tw * blk)
    dfl_nll = (logz4[0:1, :] + logz4[1:2, :]
               + logz4[2:3, :] + logz4[3:4, :]) - val
    dfl_sum = jnp.sum(fg_f * dfl_nll)

    np_sum = jnp.sum(fg_f)

    return iou_sum, cls_sum, dfl_sum, np_sum


def _loss_kernel(geo_ref, p0_ref, p1_ref, p2_ref, gtb_ref, gtbt_ref,
                 gtl_ref, out_ref):
    cx = geo_ref[0:1, :]
    cy = geo_ref[1:2, :]
    stride = geo_ref[2:3, :]
    for i in range(PB):
        x = jnp.concatenate(
            [p0_ref[i, 64:64 + C, :], p1_ref[i, 64:64 + C, :],
             p2_ref[i, 64:64 + C, :]], axis=1)
        d = jnp.concatenate(
            [p0_ref[i, 0:64, :], p1_ref[i, 0:64, :],
             p2_ref[i, 0:64, :]], axis=1)
        sums = _one_image(cx, cy, stride, x, d, gtb_ref[i], gtbt_ref[i],
                          gtl_ref[i])
        out_ref[i, 0, 0] = sums[0]
        out_ref[i, 0, 1] = sums[1]
        out_ref[i, 0, 2] = sums[2]
        out_ref[i, 0, 3] = sums[3]


@functools.partial(jax.jit, static_argnames=())
def kernel(p0, p1, p2, gt_bboxes, gt_labels):
    B = p0.shape[0]
    p0r = p0.reshape(B, 144, N0)
    p1r = p1.reshape(B, 144, N1)
    p2r = p2.reshape(B, 144, N2)
    gtbt = gt_bboxes.transpose(0, 2, 1)
    gtl = gt_labels.astype(jnp.int32).reshape(B, G, 1)
    geo = jnp.asarray(_GEO)

    parts = pl.pallas_call(
        _loss_kernel,
        grid=(B // PB,),
        in_specs=[
            pl.BlockSpec((8, N), lambda b: (0, 0)),
            pl.BlockSpec((PB, 144, N0), lambda b: (b, 0, 0)),
            pl.BlockSpec((PB, 144, N1), lambda b: (b, 0, 0)),
            pl.BlockSpec((PB, 144, N2), lambda b: (b, 0, 0)),
            pl.BlockSpec((PB, G, 4), lambda b: (b, 0, 0)),
            pl.BlockSpec((PB, 4, G), lambda b: (b, 0, 0)),
            pl.BlockSpec((PB, G, 1), lambda b: (b, 0, 0)),
        ],
        out_specs=pl.BlockSpec((PB, 1, 4), lambda b: (b, 0, 0),
                               memory_space=pltpu.SMEM),
        out_shape=jax.ShapeDtypeStruct((B, 1, 4), jnp.float32),
        compiler_params=pltpu.CompilerParams(
            dimension_semantics=("parallel",)),
    )(geo, p0r, p1r, p2r, gt_bboxes, gtbt, gtl)

    sums = parts.sum(axis=(0, 1))
    denom = jnp.maximum(1.0, sums[3])
    return jnp.stack([W_IOU * sums[0], W_CLS * sums[1],
                      W_DFL * sums[2]]) / denom


# speedup vs baseline: 1.0365x; 1.0276x over previous
import functools

import jax
import jax.numpy as jnp
import numpy as np
from jax.experimental import pallas as pl
from jax.experimental.pallas import tpu as pltpu

N0, N1, N2 = 64 * 64, 32 * 32, 16 * 16
N = N0 + N1 + N2
G = 20
C = 80
BINS = 16
TOPK = 10
EPS = 1e-9
PB = 1
W_CLS, W_IOU, W_DFL = 0.5, 7.5, 1.5


def _make_geo():
    rows = []
    for (h, w, s) in ((64, 64, 8.0), (32, 32, 16.0), (16, 16, 32.0)):
        yy, xx = np.meshgrid(np.arange(h, dtype=np.float32),
                             np.arange(w, dtype=np.float32), indexing="ij")
        cx = ((xx + 0.5) * s).reshape(-1)
        cy = ((yy + 0.5) * s).reshape(-1)
        st = np.full(h * w, s, np.float32)
        rows.append(np.stack([cx, cy, st], 0))
    geo = np.concatenate(rows, axis=1)
    return np.concatenate([geo, np.zeros((5, N), np.float32)], axis=0)


_GEO = _make_geo()

_ATAN_C = (0.9999999937538802, -0.33333137974716015, 0.19993694319366187,
           -0.14211106054436182, 0.10667486902233639, -0.07556900202632058,
           0.043278241746605356, -0.01641319040050145, 0.0029327619377836774)


def _atan(x):
    ax = jnp.abs(x)
    inv = ax > 1.0
    t = jnp.where(inv, 1.0 / jnp.maximum(ax, 1e-30), ax)
    u = t * t
    p = jnp.float32(_ATAN_C[-1])
    for c in _ATAN_C[-2::-1]:
        p = p * u + c
    r = t * p
    r = jnp.where(inv, jnp.float32(np.pi / 2) - r, r)
    return jnp.where(x < 0, -r, r)


def _mm(a, b):
    return jnp.dot(a, b, preferred_element_type=jnp.float32)


def _one_image(cx, cy, stride, x, d, gtb, gtbt, lblraw):
    bsel = jax.lax.broadcasted_iota(jnp.int32, (8, 64), 0)
    bcol = jax.lax.broadcasted_iota(jnp.int32, (8, 64), 1)
    side_of = bcol // BINS
    binv = (bcol % BINS).astype(jnp.float32)
    smat = jnp.where((bsel % 4) == side_of,
                     jnp.where(bsel < 4, 1.0, binv), 0.0)
    e = jnp.exp(d)
    red = _mm(smat, e)
    se4 = red[0:4, :]
    dist4 = red[4:8, :] / se4 * stride
    logz4 = jnp.log(se4)
    bx1 = cx - dist4[0:1, :]
    by1 = cy - dist4[1:2, :]
    bx2 = cx + dist4[2:3, :]
    by2 = cy + dist4[3:4, :]

    gx1 = gtb[:, 0:1]
    gy1 = gtb[:, 1:2]
    gx2 = gtb[:, 2:3]
    gy2 = gtb[:, 3:4]
    lbl = jnp.clip(lblraw, 0, C - 1)

    lbl_oh = (lbl == jax.lax.broadcasted_iota(jnp.int32, (G, C), 1)
              ).astype(jnp.float32)
    cls_g = _mm(lbl_oh, x)
    cls_s = jax.nn.sigmoid(cls_g)

    ix1 = jnp.maximum(gx1, bx1)
    iy1 = jnp.maximum(gy1, by1)
    ix2 = jnp.minimum(gx2, bx2)
    iy2 = jnp.minimum(gy2, by2)
    inter = jnp.clip(ix2 - ix1, 0) * jnp.clip(iy2 - iy1, 0)
    ag = (gx2 - gx1) * (gy2 - gy1)
    ap = (bx2 - bx1) * (by2 - by1)
    union = ag + ap - inter
    iou = inter / (union + EPS)

    iou2 = iou * iou
    iou6 = iou2 * iou2 * iou2
    align = jnp.sqrt(cls_s) * iou6

    pcx = (bx1 + bx2) * 0.5
    pcy = (by1 + by2) * 0.5
    in_gt = (pcx >= gx1) & (pcx < gx2) & (pcy >= gy1) & (pcy < gy2)
    valid = ((gx2 - gx1) > 0) & ((gy2 - gy1) > 0)
    mask = in_gt & valid
    metric = jnp.where(mask, align, 0.0)

    mwork = metric
    mv = None
    for _ in range(TOPK):
        mv = jnp.max(mwork, axis=1, keepdims=True)
        mwork = jnp.where(mwork == mv, -1.0, mwork)
    mp = ((metric >= mv) & (metric > EPS)).astype(jnp.float32)

    ones_g = jnp.ones((1, G), jnp.float32)
    g_iota = jax.lax.broadcasted_iota(jnp.int32, (G, N), 0)
    g_col = jax.lax.broadcasted_iota(jnp.int32, (G, G), 1)
    g_row = jax.lax.broadcasted_iota(jnp.int32, (G, G), 0)
    tril = (g_col <= g_row).astype(jnp.float32)

    fg_count = _mm(ones_g, mp)

    x_iou = jnp.where(mp > 0, iou, -1.0)
    mxv = jnp.max(x_iou, axis=0, keepdims=True)
    eqm = (x_iou == mxv).astype(jnp.float32)
    is_max = jnp.where(_mm(tril, eqm) == 1.0, eqm, 0.0)
    mp = jnp.where(fg_count > 1, is_max * mp, mp)

    fg_f = jnp.minimum(_mm(ones_g, mp), 1.0)
    cum = _mm(tril, mp)
    moh = jnp.where((mp > 0) & (cum == 1.0), 1.0, 0.0)
    moh = moh + (1.0 - fg_f) * (g_iota == 0).astype(jnp.float32)

    tboxes = _mm(gtbt, moh)
    tx1 = tboxes[0:1, :]
    ty1 = tboxes[1:2, :]
    tx2 = tboxes[2:3, :]
    ty2 = tboxes[3:4, :]
    xsel = _mm(ones_g, moh * cls_g)

    align_m = align * mp
    pos_align = jnp.max(align_m, axis=1, keepdims=True)
    pos_iou = jnp.max(iou * mp, axis=1, keepdims=True)
    norm = jnp.max(align_m * pos_iou / (pos_align + EPS), axis=0,
                   keepdims=True)
    w = norm * fg_f

    ones_c = jnp.ones((1, C), jnp.float32)
    t = jnp.abs(x)
    l1p = jnp.log1p(jnp.exp(-t))
    base = (0.5 * jnp.sum(_mm(ones_c, x + t))
            + jnp.sum(_mm(ones_c, l1p)))
    cls_sum = base - jnp.sum(w * xsel)

    def _pk(a):
        return a.reshape(N // 128, 128)

    pbx1 = _pk(bx1)
    pby1 = _pk(by1)
    pbx2 = _pk(bx2)
    pby2 = _pk(by2)
    ptx1 = _pk(tx1)
    pty1 = _pk(ty1)
    ptx2 = _pk(tx2)
    pty2 = _pk(ty2)
    pfg = _pk(fg_f)
    pap = _pk(ap)
    cix1 = jnp.maximum(pbx1, ptx1)
    ciy1 = jnp.maximum(pby1, pty1)
    cix2 = jnp.minimum(pbx2, ptx2)
    ciy2 = jnp.minimum(pby2, pty2)
    cinter = jnp.clip(cix2 - cix1, 0) * jnp.clip(ciy2 - ciy1, 0)
    at = (ptx2 - ptx1) * (pty2 - pty1)
    cunion = pap + at - cinter
    ciou = cinter / (cunion + EPS)
    ex1 = jnp.minimum(pbx1, ptx1)
    ey1 = jnp.minimum(pby1, pty1)
    ex2 = jnp.maximum(pbx2, ptx2)
    ey2 = jnp.maximum(pby2, pty2)
    c2 = (ex2 - ex1) ** 2 + (ey2 - ey1) ** 2 + EPS
    rho2 = ((pbx1 + pbx2 - ptx1 - ptx2) ** 2
            + (pby1 + pby2 - pty1 - pty2) ** 2) / 4.0
    wp = pbx2 - pbx1
    hp = pby2 - pby1 + EPS
    wt = ptx2 - ptx1
    ht = pty2 - pty1 + EPS
    v = (4.0 / (np.pi ** 2)) * (_atan(wt / ht) - _atan(wp / hp)) ** 2
    a = v / (v - ciou + 1.0 + EPS)
    lci = 1.0 - (ciou - rho2 / c2 - a * v)
    iou_sum = jnp.sum(lci * pfg)

    tdists = (jnp.clip(cx - tx1, 0), jnp.clip(cy - ty1, 0),
              jnp.clip(tx2 - cx, 0), jnp.clip(ty2 - cy, 0))
    b_iota = jax.lax.broadcasted_iota(
        jnp.int32, (BINS, N), 0).astype(jnp.float32)
    ones_b = jnp.ones((1, BINS), jnp.float32)
    val = jnp.zeros((1, N), jnp.float32)
    for s in range(4):
        tb = jnp.clip(tdists[s], 0.0, BINS - 1 - 1e-6)
        blk = d[BINS * s:BINS * (s + 1), :]
        tw = jnp.maximum(1.0 - jnp.abs(tb - b_iota), 0.0)
        val = val + _mm(ones_b, tw * blk)
    dfl_nll = (logz4[0:1, :] + logz4[1:2, :]
               + logz4[2:3, :] + logz4[3:4, :]) - val
    dfl_sum = jnp.sum(fg_f * dfl_nll)

    np_sum = jnp.sum(fg_f)

    return iou_sum, cls_sum, dfl_sum, np_sum


def _loss_kernel(geo_ref, p0_ref, p1_ref, p2_ref, gtb_ref, gtbt_ref,
                 gtl_ref, out_ref):
    cx = geo_ref[0:1, :]
    cy = geo_ref[1:2, :]
    stride = geo_ref[2:3, :]
    for i in range(PB):
        x = jnp.concatenate(
            [p0_ref[i, 64:64 + C, :], p1_ref[i, 64:64 + C, :],
             p2_ref[i, 64:64 + C, :]], axis=1)
        d = jnp.concatenate(
            [p0_ref[i, 0:64, :], p1_ref[i, 0:64, :],
             p2_ref[i, 0:64, :]], axis=1)
        sums = _one_image(cx, cy, stride, x, d, gtb_ref[i], gtbt_ref[i],
                          gtl_ref[i])
        out_ref[i, 0, 0] = sums[0]
        out_ref[i, 0, 1] = sums[1]
        out_ref[i, 0, 2] = sums[2]
        out_ref[i, 0, 3] = sums[3]


@functools.partial(jax.jit, static_argnames=())
def kernel(p0, p1, p2, gt_bboxes, gt_labels):
    B = p0.shape[0]
    p0r = p0.reshape(B, 144, N0)
    p1r = p1.reshape(B, 144, N1)
    p2r = p2.reshape(B, 144, N2)
    gtbt = gt_bboxes.transpose(0, 2, 1)
    gtl = gt_labels.astype(jnp.int32).reshape(B, G, 1)
    geo = jnp.asarray(_GEO)

    parts = pl.pallas_call(
        _loss_kernel,
        grid=(B // PB,),
        in_specs=[
            pl.BlockSpec((8, N), lambda b: (0, 0)),
            pl.BlockSpec((PB, 144, N0), lambda b: (b, 0, 0)),
            pl.BlockSpec((PB, 144, N1), lambda b: (b, 0, 0)),
            pl.BlockSpec((PB, 144, N2), lambda b: (b, 0, 0)),
            pl.BlockSpec((PB, G, 4), lambda b: (b, 0, 0)),
            pl.BlockSpec((PB, 4, G), lambda b: (b, 0, 0)),
            pl.BlockSpec((PB, G, 1), lambda b: (b, 0, 0)),
        ],
        out_specs=pl.BlockSpec((PB, 1, 4), lambda b: (b, 0, 0),
                               memory_space=pltpu.SMEM),
        out_shape=jax.ShapeDtypeStruct((B, 1, 4), jnp.float32),
        compiler_params=pltpu.CompilerParams(
            dimension_semantics=("parallel",)),
    )(geo, p0r, p1r, p2r, gt_bboxes, gtbt, gtl)

    sums = parts.sum(axis=(0, 1))
    denom = jnp.maximum(1.0, sums[3])
    return jnp.stack([W_IOU * sums[0], W_CLS * sums[1],
                      W_DFL * sums[2]]) / denom
